# Initial kernel scaffold; baseline (speedup 1.0000x reference)
import jax
import jax.numpy as jnp
from jax.experimental import pallas as pl


def kernel(idx_patient, idx_tissue, idx_metagene, hyperedge_attr, patient_feat, tissue_feat, metagene_feat, edge_W0, edge_b0, edge_W1, edge_b1, edge_W2, edge_b2, att_W0, att_W1, upd_W0, upd_b0, upd_W1, upd_b1, upd_W2, upd_b2):
    raise NotImplementedError("write your pallas kernel here")


# speedup vs baseline: 2.3688x; 2.3688x over previous
import jax
import jax.numpy as jnp
from jax import lax
from jax.experimental import pallas as pl
from jax.experimental.pallas import tpu as pltpu
from jax.experimental.pallas import tpu_sc as plsc

NP = 50000
NT = 64
NM = 300
E = 800000
DP = 64
MSG = 80
H = 5
MT = NM * NT

NC = 2
NS = 16
NWORK = NC * NS
CHUNK = 128
NCHUNKS = E // CHUNK
CH_PER_W_LO = NCHUNKS // NWORK
CH_EXTRA_W = NCHUNKS - CH_PER_W_LO * NWORK

DEN_ROWS = 50048


def _worker_chunks(wid):
    return jnp.where(wid < CH_EXTRA_W, CH_PER_W_LO + 1, CH_PER_W_LO)


def _patproj_body(x_ref, w_ref, b_ref, o_ref):
    o_ref[...] = jnp.dot(x_ref[...], w_ref[...],
                         preferred_element_type=jnp.float32) + b_ref[...]


def _tables_body(met_ref, tis_ref, w0m_ref, w0t_ref, am_ref, at_ref,
                 aw1_ref, p5_ref, mt_ref):
    met = met_ref[...]
    tis = tis_ref[...]
    met0 = jnp.dot(met, w0m_ref[...], preferred_element_type=jnp.float32)
    tis0 = jnp.dot(tis, w0t_ref[...], preferred_element_type=jnp.float32)
    mt0 = (met0[:, None, :] + tis0[None, :, :]).reshape(MT, 128)
    matt = jnp.dot(met, am_ref[...], preferred_element_type=jnp.float32)
    tatt = jnp.dot(tis, at_ref[...], preferred_element_type=jnp.float32)
    s = matt[:, None, :] + tatt[None, :, :]
    s = jnp.where(s > 0, s, 0.1 * s).reshape(MT, 64)
    logits = jnp.dot(s, aw1_ref[...], preferred_element_type=jnp.float32)
    gmax = jnp.max(logits, axis=0, keepdims=True)
    w5 = jnp.exp(logits - gmax)
    mt_ref[...] = mt0 + jnp.dot(w5, p5_ref[...],
                                preferred_element_type=jnp.float32)


def _sc_gather_body(idxp_hbm, idxt_hbm, idxm_hbm, patproj_hbm, mt_hbm,
                    z16_hbm,
                    hw_hbm, denp_hbm,
                    idxp_v, idxt_v, idxm_v, idxmt_v, pat_v, mt_v, w_v,
                    den_sh, sem):
    c = lax.axis_index("c")
    s = lax.axis_index("s")
    wid = c * NS + s

    pltpu.sync_copy(z16_hbm, den_sh.at[pl.ds(s * 3128, 3128), :])
    plsc.subcore_barrier()

    def chunk_body(i, carry):
        start = (wid + i * NWORK) * CHUNK
        pltpu.sync_copy(idxp_hbm.at[pl.ds(start, CHUNK)], idxp_v)
        pltpu.sync_copy(idxt_hbm.at[pl.ds(start, CHUNK)], idxt_v)
        pltpu.sync_copy(idxm_hbm.at[pl.ds(start, CHUNK)], idxm_v)

        def mt_body(k, carry2):
            kk = k * 16
            idxmt_v[pl.ds(kk, 16)] = (idxm_v[pl.ds(kk, 16)] * NT
                                      + idxt_v[pl.ds(kk, 16)])
            return carry2
        lax.fori_loop(0, CHUNK // 16, mt_body, 0)

        pltpu.async_copy(patproj_hbm.at[idxp_v], pat_v, sem).wait()
        pltpu.async_copy(mt_hbm.at[idxmt_v], mt_v, sem).wait()

        def row_body(r, carry2):
            for k in range(128 // 16):
                kk = k * 16
                pat_v[r, pl.ds(kk, 16)] = (pat_v[r, pl.ds(kk, 16)]
                                           + mt_v[r, pl.ds(kk, 16)])
            w_v[r, pl.ds(0, 16)] = pat_v[r, pl.ds(MSG, 16)]
            return carry2
        lax.fori_loop(0, CHUNK, row_body, 0)

        pltpu.sync_copy(pat_v, hw_hbm.at[pl.ds(start, CHUNK), :])
        pltpu.sync_copy(w_v, den_sh.at[idxp_v], add=True)
        return carry

    lax.fori_loop(0, _worker_chunks(wid), chunk_body, 0)
    plsc.subcore_barrier()

    row0 = s * 3128

    def flush_body(j, carry):
        r0 = row0 + j * CHUNK
        pltpu.sync_copy(den_sh.at[pl.ds(r0, CHUNK), :], w_v)
        pltpu.sync_copy(w_v, denp_hbm.at[c, pl.ds(r0, CHUNK), :])
        return carry
    lax.fori_loop(0, 24, flush_body, 0)
    r0 = row0 + 24 * CHUNK
    pltpu.sync_copy(den_sh.at[pl.ds(r0, 56), :], w_v.at[pl.ds(0, 56), :])
    pltpu.sync_copy(w_v.at[pl.ds(0, 56), :], denp_hbm.at[c, pl.ds(r0, 56), :])


def _edge_mlp_body(hw_ref, attr_ref, w0a_ref, w1_ref, b1_ref,
                   w2_ref, b2_ref, rm_ref, o0_ref, o1_ref, o2_ref,
                   o3_ref, o4_ref):
    def ln(x):
        mu = jnp.mean(x, axis=-1, keepdims=True)
        var = jnp.var(x, axis=-1, keepdims=True)
        return (x - mu) / jnp.sqrt(var + 1e-5)

    hw = hw_ref[...]
    h0 = hw[:, :MSG] + jnp.dot(attr_ref[...], w0a_ref[...],
                               preferred_element_type=jnp.float32)
    x = jax.nn.relu(ln(h0))
    x = jnp.dot(x, w1_ref[...], preferred_element_type=jnp.float32) + b1_ref[...]
    x = jax.nn.relu(ln(x))
    msg = jnp.dot(x, w2_ref[...], preferred_element_type=jnp.float32) + b2_ref[...]
    wrep = jnp.dot(hw[:, MSG:MSG + H], rm_ref[...],
                   preferred_element_type=jnp.float32)
    wmsg = msg * wrep
    o0_ref[...] = wmsg[:, 0:16]
    o1_ref[...] = wmsg[:, 16:32]
    o2_ref[...] = wmsg[:, 32:48]
    o3_ref[...] = wmsg[:, 48:64]
    o4_ref[...] = wmsg[:, 64:80]


def _sc_scatter_body(idxp_hbm, w0_hbm, w1_hbm, w2_hbm, w3_hbm, w4_hbm,
                     z16_hbm, up_hbm,
                     idx_v, wm_v, u_sh, sem):
    c = lax.axis_index("c")
    s = lax.axis_index("s")
    wid = c * NS + s
    nch = _worker_chunks(wid)

    for h, wh_hbm in enumerate((w0_hbm, w1_hbm, w2_hbm, w3_hbm, w4_hbm)):
        pltpu.sync_copy(z16_hbm, u_sh.at[pl.ds(s * 3128, 3128), :])
        plsc.subcore_barrier()

        def chunk_body(i, carry):
            start = (wid + i * NWORK) * CHUNK
            pltpu.sync_copy(idxp_hbm.at[pl.ds(start, CHUNK)], idx_v)
            pltpu.sync_copy(wh_hbm.at[pl.ds(start, CHUNK), :], wm_v)
            pltpu.sync_copy(wm_v, u_sh.at[idx_v], add=True)
            return carry
        lax.fori_loop(0, nch, chunk_body, 0)
        plsc.subcore_barrier()

        row0 = s * 3128

        def flush_body(j, carry):
            r0 = row0 + j * CHUNK
            pltpu.sync_copy(u_sh.at[pl.ds(r0, CHUNK), :], wm_v)
            pltpu.sync_copy(wm_v, up_hbm.at[c, h, pl.ds(r0, CHUNK), :])
            return carry
        lax.fori_loop(0, 24, flush_body, 0)
        r0 = row0 + 24 * CHUNK
        pltpu.sync_copy(u_sh.at[pl.ds(r0, 56), :], wm_v.at[pl.ds(0, 56), :])
        pltpu.sync_copy(wm_v.at[pl.ds(0, 56), :],
                        up_hbm.at[c, h, pl.ds(r0, 56), :])
        plsc.subcore_barrier()


def _update_body(pat_ref, d0_ref, d1_ref,
                 u00_ref, u01_ref, u02_ref, u03_ref, u04_ref,
                 u10_ref, u11_ref, u12_ref, u13_ref, u14_ref,
                 w0_ref, b0_ref, w1_ref, b1_ref, w2_ref, b2_ref, o_ref):
    def ln(x):
        mu = jnp.mean(x, axis=-1, keepdims=True)
        var = jnp.var(x, axis=-1, keepdims=True)
        return (x - mu) / jnp.sqrt(var + 1e-5)

    den = d0_ref[...] + d1_ref[...]
    inv = 1.0 / (den[:, :H] + 1e-16)
    u0 = (u00_ref[...] + u10_ref[...]) * inv[:, 0:1]
    u1 = (u01_ref[...] + u11_ref[...]) * inv[:, 1:2]
    u2 = (u02_ref[...] + u12_ref[...]) * inv[:, 2:3]
    u3 = (u03_ref[...] + u13_ref[...]) * inv[:, 3:4]
    u4 = (u04_ref[...] + u14_ref[...]) * inv[:, 4:5]
    pat = pat_ref[...]
    cat = jnp.concatenate([pat, u0, u1, u2, u3, u4], axis=-1)
    x = jnp.dot(cat, w0_ref[...], preferred_element_type=jnp.float32) + b0_ref[...]
    x = jax.nn.relu(ln(x))
    x = jnp.dot(x, w1_ref[...], preferred_element_type=jnp.float32) + b1_ref[...]
    x = jax.nn.relu(ln(x))
    o_ref[...] = pat + jnp.dot(x, w2_ref[...],
                               preferred_element_type=jnp.float32) + b2_ref[...]


def kernel(idx_patient, idx_tissue, idx_metagene, hyperedge_attr, patient_feat,
           tissue_feat, metagene_feat, edge_W0, edge_b0, edge_W1, edge_b1,
           edge_W2, edge_b2, att_W0, att_W1, upd_W0, upd_b0, upd_W1, upd_b1,
           upd_W2, upd_b2):
    f32 = jnp.float32
    idx_patient = idx_patient.astype(jnp.int32)
    idx_tissue = idx_tissue.astype(jnp.int32)
    idx_metagene = idx_metagene.astype(jnp.int32)

    W0_met = edge_W0[0:32]
    W0_pat = edge_W0[32:96]
    W0_tis = edge_W0[96:128]
    W0_attr = edge_W0[128:132]

    p5 = (jnp.arange(H)[:, None] + MSG == jnp.arange(128)[None, :]).astype(f32)
    rm = (jnp.arange(MSG)[None, :] // 16 == jnp.arange(H)[:, None]).astype(f32)
    z16 = jnp.zeros((3128, 16), f32)
    pad48 = jnp.zeros((32, 128 - MSG), f32)
    W0m_p = jnp.concatenate([W0_met, pad48], axis=1)
    W0t_p = jnp.concatenate([W0_tis, pad48], axis=1)
    W0p_p = jnp.concatenate([W0_pat, jnp.zeros((DP, 128 - MSG), f32)], axis=1)
    b0_p = jnp.concatenate([edge_b0, jnp.zeros((128 - MSG,), f32)])

    pat_proj = pl.pallas_call(
        _patproj_body,
        grid=(10,),
        in_specs=[pl.BlockSpec((5000, DP), lambda i: (i, 0)),
                  pl.BlockSpec((DP, 128), lambda i: (0, 0)),
                  pl.BlockSpec((1, 128), lambda i: (0, 0))],
        out_specs=pl.BlockSpec((5000, 128), lambda i: (i, 0)),
        out_shape=jax.ShapeDtypeStruct((NP, 128), f32),
    )(patient_feat, W0p_p, b0_p.reshape(1, 128))

    mt_tab = pl.pallas_call(
        _tables_body,
        out_shape=jax.ShapeDtypeStruct((MT, 128), f32),
    )(metagene_feat, tissue_feat, W0m_p, W0t_p, att_W0[:32], att_W0[32:],
      att_W1, p5)

    mesh = plsc.VectorSubcoreMesh(core_axis_name="c", subcore_axis_name="s",
                                  num_cores=NC, num_subcores=NS)
    hw, den_part = pl.kernel(
        _sc_gather_body,
        out_type=[jax.ShapeDtypeStruct((E, 128), f32),
                  jax.ShapeDtypeStruct((NC, DEN_ROWS, 16), f32)],
        mesh=mesh,
        compiler_params=pltpu.CompilerParams(use_tc_tiling_on_sc=False),
        scratch_types=[pltpu.VMEM((CHUNK,), jnp.int32),
                       pltpu.VMEM((CHUNK,), jnp.int32),
                       pltpu.VMEM((CHUNK,), jnp.int32),
                       pltpu.VMEM((CHUNK,), jnp.int32),
                       pltpu.VMEM((CHUNK, 128), f32),
                       pltpu.VMEM((CHUNK, 128), f32),
                       pltpu.VMEM((CHUNK, 16), f32),
                       pltpu.VMEM_SHARED((DEN_ROWS, 16), f32),
                       pltpu.SemaphoreType.DMA],
    )(idx_patient, idx_tissue, idx_metagene, pat_proj, mt_tab, z16)

    BB = 6400
    wmsg_h = pl.pallas_call(
        _edge_mlp_body,
        grid=(E // BB,),
        in_specs=[pl.BlockSpec((BB, 128), lambda i: (i, 0)),
                  pl.BlockSpec((BB, 4), lambda i: (i, 0)),
                  pl.BlockSpec((4, MSG), lambda i: (0, 0)),
                  pl.BlockSpec((MSG, MSG), lambda i: (0, 0)),
                  pl.BlockSpec((1, MSG), lambda i: (0, 0)),
                  pl.BlockSpec((MSG, MSG), lambda i: (0, 0)),
                  pl.BlockSpec((1, MSG), lambda i: (0, 0)),
                  pl.BlockSpec((H, MSG), lambda i: (0, 0))],
        out_specs=[pl.BlockSpec((BB, 16), lambda i: (i, 0))] * H,
        out_shape=[jax.ShapeDtypeStruct((E, 16), f32)] * H,
    )(hw, hyperedge_attr, W0_attr, edge_W1, edge_b1.reshape(1, MSG),
      edge_W2, edge_b2.reshape(1, MSG), rm)

    u_part = pl.kernel(
        _sc_scatter_body,
        out_type=[jax.ShapeDtypeStruct((NC, H, DEN_ROWS, 16), f32)],
        mesh=plsc.VectorSubcoreMesh(core_axis_name="c", subcore_axis_name="s",
                                    num_cores=NC, num_subcores=NS),
        compiler_params=pltpu.CompilerParams(use_tc_tiling_on_sc=False),
        scratch_types=[pltpu.VMEM((CHUNK,), jnp.int32),
                       pltpu.VMEM((CHUNK, 16), f32),
                       pltpu.VMEM_SHARED((DEN_ROWS, 16), f32),
                       pltpu.SemaphoreType.DMA],
    )(idx_patient, wmsg_h[0], wmsg_h[1], wmsg_h[2], wmsg_h[3], wmsg_h[4],
      z16)[0]

    d0 = den_part[0, :NP, :]
    d1 = den_part[1, :NP, :]
    us = [u_part[cc, hh, :NP, :] for cc in range(NC) for hh in range(H)]

    UPD_IN = DP + MSG
    spec16 = pl.BlockSpec((1000, 16), lambda i: (i, 0))
    patient_out = pl.pallas_call(
        _update_body,
        grid=(50,),
        in_specs=[pl.BlockSpec((1000, DP), lambda i: (i, 0)),
                  spec16, spec16,
                  spec16, spec16, spec16, spec16, spec16,
                  spec16, spec16, spec16, spec16, spec16,
                  pl.BlockSpec((UPD_IN, DP), lambda i: (0, 0)),
                  pl.BlockSpec((1, DP), lambda i: (0, 0)),
                  pl.BlockSpec((DP, DP), lambda i: (0, 0)),
                  pl.BlockSpec((1, DP), lambda i: (0, 0)),
                  pl.BlockSpec((DP, DP), lambda i: (0, 0)),
                  pl.BlockSpec((1, DP), lambda i: (0, 0))],
        out_specs=pl.BlockSpec((1000, DP), lambda i: (i, 0)),
        out_shape=jax.ShapeDtypeStruct((NP, DP), f32),
    )(patient_feat, d0, d1, *us, upd_W0, upd_b0.reshape(1, DP),
      upd_W1, upd_b1.reshape(1, DP), upd_W2, upd_b2.reshape(1, DP))

    return (patient_out, hyperedge_attr)
